# Initial kernel scaffold; baseline (speedup 1.0000x reference)
import jax
import jax.numpy as jnp
from jax.experimental import pallas as pl


def kernel(indices, table, W, b):
    raise NotImplementedError("write your pallas kernel here")


# speedup vs baseline: 3.4368x; 3.4368x over previous
import functools

import jax
import jax.numpy as jnp
from jax import lax
from jax.experimental import pallas as pl
from jax.experimental.pallas import tpu as pltpu
from jax.experimental.pallas import tpu_sc as plsc

_VOCAB = 37
_PAIR_W = 2 * _VOCAB
_PAIR_ROWS = _VOCAB * _VOCAB
_NUM_CORES = 2
_NUM_SUBCORES = 16
_NW = _NUM_CORES * _NUM_SUBCORES
_CH = 128
_DEPTH = 4
_LANES = 16


def _pair_table_body(table_ref, w_ref, b_ref, p2_ref):
    p = lax.dot_general(
        table_ref[...], w_ref[...], (((1,), (1,)), ((), ())),
        preferred_element_type=jnp.float32)
    p = p + b_ref[...]
    left = jnp.broadcast_to(p[:, None, :], (_VOCAB, _VOCAB, _VOCAB))
    right = jnp.broadcast_to(p[None, :, :], (_VOCAB, _VOCAB, _VOCAB))
    p2_ref[...] = jnp.concatenate([left, right], axis=-1)


def _build_pair_table(table, W, b):
    return pl.pallas_call(
        _pair_table_body,
        out_shape=jax.ShapeDtypeStruct((_VOCAB, _VOCAB, _PAIR_W), jnp.float32),
    )(table, W, b.reshape(1, _VOCAB))


def _make_sc_gather(n_pairs):
    rows_per_tile = n_pairs // _NW
    assert rows_per_tile * _NW == n_pairs
    assert rows_per_tile % _CH == 0 and rows_per_tile % _LANES == 0
    nch = rows_per_tile // _CH
    mesh = plsc.VectorSubcoreMesh(
        core_axis_name="c", subcore_axis_name="s")

    scratch = [
        pltpu.VMEM((2 * rows_per_tile,), jnp.int32),
        pltpu.VMEM((rows_per_tile,), jnp.int32),
    ]
    scratch += [pltpu.VMEM((_CH, _PAIR_W), jnp.float32) for _ in range(_DEPTH)]
    scratch += [pltpu.SemaphoreType.DMA for _ in range(2 * _DEPTH)]

    @functools.partial(
        pl.kernel,
        out_type=jax.ShapeDtypeStruct((n_pairs, _PAIR_W), jnp.float32),
        mesh=mesh,
        scratch_types=scratch,
        compiler_params=pltpu.CompilerParams(
            needs_layout_passes=False, use_tc_tiling_on_sc=False),
    )
    def sc_gather(p2_hbm, idx_hbm, out_hbm, idx_v, pi_v, *rest):
        bufs = rest[:_DEPTH]
        gsems = rest[_DEPTH:2 * _DEPTH]
        ssems = rest[2 * _DEPTH:]
        wid = lax.axis_index("s") * _NUM_CORES + lax.axis_index("c")
        row0 = wid * rows_per_tile

        pltpu.sync_copy(idx_hbm.at[pl.ds(row0 * 2, 2 * rows_per_tile)], idx_v)

        lane = lax.iota(jnp.int32, _LANES)

        def compute_pi(j, carry):
            base = j * (2 * _LANES)
            ev = plsc.load_gather(idx_v, [lane * 2 + base])
            od = plsc.load_gather(idx_v, [lane * 2 + base + 1])
            pi_v[pl.ds(j * _LANES, _LANES)] = ev * _VOCAB + od
            return carry

        lax.fori_loop(0, rows_per_tile // _LANES, compute_pi, 0)

        gcopies = [None] * _DEPTH
        scopies = [None] * _DEPTH
        for c in range(nch + 1):
            if c < nch:
                sl = c % _DEPTH
                if scopies[sl] is not None:
                    scopies[sl].wait()
                gcopies[sl] = pltpu.async_copy(
                    p2_hbm.at[pi_v.at[pl.ds(c * _CH, _CH)]], bufs[sl],
                    gsems[sl])
            c2 = c - 1
            if c2 >= 0:
                sl2 = c2 % _DEPTH
                gcopies[sl2].wait()
                scopies[sl2] = pltpu.async_copy(
                    bufs[sl2], out_hbm.at[pl.ds(row0 + c2 * _CH, _CH)],
                    ssems[sl2])
        for c2 in range(max(0, nch - _DEPTH), nch):
            scopies[c2 % _DEPTH].wait()

    return sc_gather


def kernel(indices, table, W, b):
    bsz, seqlen = indices.shape
    p2 = _build_pair_table(table, W, b).reshape(_PAIR_ROWS, _PAIR_W)
    idx_flat = indices.reshape(-1)
    n_pairs = idx_flat.shape[0] // 2
    out2 = _make_sc_gather(n_pairs)(p2, idx_flat)
    return out2.reshape(bsz, seqlen, _VOCAB)
